# baseline (speedup 1.0000x reference)
import jax
import jax.numpy as jnp
from jax.experimental import pallas as pl
from jax.experimental.pallas import tpu as pltpu

_B, _N, _M = 8, 2048, 2048
_TN = 256
_BS = 8

_DN = (((0,), (0,)), ((), ()))


def _chamfer_body(tsT_ref, out_ref, ta_ref, sa_ref):
    b = pl.program_id(0)
    bf = jnp.bfloat16

    @pl.when(b == 0)
    def _assemble():
        for bb in range(_B):
            base = bb * 16
            tt = tsT_ref[bb]
            t2 = tt[0:1, :] * tt[0:1, :] + tt[1:2, :] * tt[1:2, :] \
                + tt[2:3, :] * tt[2:3, :]
            t2_hi = t2.astype(bf)
            t2_lo = (t2 - t2_hi.astype(jnp.float32)).astype(bf)
            ta_ref[base : base + 3, :] = tt.astype(bf)
            ta_ref[base + 3 : base + 5, :] = jnp.ones((2, _N), dtype=bf)
            ta_ref[base + 5 : base + 6, :] = t2_hi
            ta_ref[base + 6 : base + 7, :] = t2_lo
            ta_ref[base + 7 : base + 8, :] = jnp.zeros((1, _N), dtype=bf)

            st = tsT_ref[_B + bb]
            s2 = st[0:1, :] * st[0:1, :] + st[1:2, :] * st[1:2, :] \
                + st[2:3, :] * st[2:3, :]
            s2_hi = s2.astype(bf)
            s2_lo = (s2 - s2_hi.astype(jnp.float32)).astype(bf)
            sa_ref[base : base + 3, :] = st.astype(bf) * bf(-2.0)
            sa_ref[base + 3 : base + 4, :] = s2_hi
            sa_ref[base + 4 : base + 5, :] = s2_lo
            sa_ref[base + 5 : base + 7, :] = jnp.ones((2, _M), dtype=bf)
            sa_ref[base + 7 : base + 8, :] = jnp.zeros((1, _M), dtype=bf)

    total = jnp.float32(0.0)
    for bb in range(_BS):
        bi = b * _BS + bb
        sa = sa_ref[pl.ds(bi * 16, 8), :]
        pres = []
        col_min = jnp.full((1, _M), jnp.inf, dtype=jnp.float32)
        for i in range(_N // _TN):
            ta = ta_ref[pl.ds(bi * 16, 8), pl.ds(i * _TN, _TN)]
            e = jax.lax.dot_general(ta, sa, _DN,
                                    preferred_element_type=jnp.float32)
            pres.append(jnp.min(e, axis=1, keepdims=True))
            col_min = jnp.minimum(col_min, jnp.min(e, axis=0, keepdims=True))

        row_min = jnp.concatenate(pres, axis=1)
        total = total + jnp.sum(jnp.sqrt(jnp.maximum(row_min, 0.0)))
        total = total + jnp.sum(jnp.sqrt(jnp.maximum(col_min, 0.0)))

    @pl.when(b == 0)
    def _():
        out_ref[...] = jnp.zeros((1, 1), dtype=jnp.float32)

    out_ref[...] += jnp.broadcast_to(total * (0.5 / (_B * _N)), (1, 1))


def kernel(template, source):
    tsT = jnp.swapaxes(jnp.concatenate([template, source], axis=0), 1, 2)
    out = pl.pallas_call(
        _chamfer_body,
        grid=(_B // _BS,),
        in_specs=[
            pl.BlockSpec((2 * _B, 3, _N), lambda b: (0, 0, 0)),
        ],
        out_specs=pl.BlockSpec((1, 1), lambda b: (0, 0)),
        out_shape=jax.ShapeDtypeStruct((1, 1), jnp.float32),
        scratch_shapes=[
            pltpu.VMEM((_B * 16, _N), jnp.bfloat16),
            pltpu.VMEM((_B * 16, _M), jnp.bfloat16),
        ],
    )(tsT)
    return out[0, 0]
